# baseline (speedup 1.0000x reference)
import functools

import jax
import jax.numpy as jnp
from jax import lax
from jax.experimental import pallas as pl
from jax.experimental.pallas import tpu as pltpu
from jax.experimental.pallas import tpu_sc as plsc

BATCH = 1024
EMBED = 768
PARTITION = 96
CENTROIDS = 256
DSUB = EMBED // PARTITION
PGROUP = 16

_CONTRACT_T = (((1,), (1,)), ((), ()))

NCORES = 2
NSUB = 16
NW = NCORES * NSUB
ROWS = BATCH * PARTITION
CHUNK = 128
NCHUNK = ROWS // (NW * CHUNK)


def _dot(a, b, dn):
    return jax.lax.dot_general(a, b, dn, preferred_element_type=jnp.float32)


def _idx_body(v_ref, cb_ref, idx_ref, cbdh_ref, cbdl_ref):
    step = pl.program_id(0)

    @pl.when(step == 0)
    def _zero():
        cbdh_ref[...] = jnp.zeros_like(cbdh_ref)
        cbdl_ref[...] = jnp.zeros_like(cbdl_ref)

    for g in range(PGROUP):
        slab = cb_ref[g]
        hi = slab.astype(jnp.bfloat16)
        lo = (slab - hi.astype(jnp.float32)).astype(jnp.bfloat16)
        rows = slice(g * CENTROIDS, (g + 1) * CENTROIDS)
        cols = slice(g * DSUB, (g + 1) * DSUB)
        cbdh_ref[rows, cols] = hi
        cbdl_ref[rows, cols] = lo

    v = v_ref[...]
    vh = v.astype(jnp.bfloat16)
    vl = (v - vh.astype(jnp.float32)).astype(jnp.bfloat16)

    s = (_dot(vh, cbdh_ref[...], _CONTRACT_T) + _dot(vh, cbdl_ref[...], _CONTRACT_T)
         + _dot(vl, cbdh_ref[...], _CONTRACT_T))

    iota = lax.broadcasted_iota(jnp.int32, (BATCH, CENTROIDS), 1)
    base0 = (step * PGROUP) * CENTROIDS
    cols = []
    for g in range(PGROUP):
        c_p = cb_ref[g]
        csq = 0.5 * jnp.sum(c_p * c_p, axis=-1)
        sg = s[:, g * CENTROIDS:(g + 1) * CENTROIDS] - csq[None, :]
        m = jnp.max(sg, axis=1, keepdims=True)
        idx = jnp.min(jnp.where(sg == m, iota, CENTROIDS), axis=1,
                      keepdims=True)
        cols.append(idx + (base0 + g * CENTROIDS))
    idx_ref[0] = jnp.concatenate(cols, axis=1)


def _tc_indices(vecs, codebook):
    ngrp = PARTITION // PGROUP
    return pl.pallas_call(
        _idx_body,
        grid=(ngrp,),
        in_specs=[
            pl.BlockSpec((BATCH, PGROUP * DSUB), lambda i: (0, i)),
            pl.BlockSpec((PGROUP, CENTROIDS, DSUB), lambda i: (i, 0, 0)),
        ],
        out_specs=pl.BlockSpec((1, BATCH, PGROUP), lambda i: (i, 0, 0)),
        out_shape=jax.ShapeDtypeStruct((PARTITION // PGROUP, BATCH, PGROUP),
                                       jnp.int32),
        scratch_shapes=[
            pltpu.VMEM((PGROUP * CENTROIDS, PGROUP * DSUB), jnp.bfloat16),
            pltpu.VMEM((PGROUP * CENTROIDS, PGROUP * DSUB), jnp.bfloat16),
        ],
    )(vecs, codebook)


@functools.partial(
    pl.kernel,
    mesh=plsc.VectorSubcoreMesh(core_axis_name="c", subcore_axis_name="s"),
    compiler_params=pltpu.CompilerParams(use_tc_tiling_on_sc=False),
    out_type=jax.ShapeDtypeStruct((NW * NCHUNK, CHUNK, DSUB), jnp.float32),
    scratch_types=[
        pltpu.VMEM((NCHUNK, CHUNK), jnp.int32),
        pltpu.VMEM((NCHUNK, CHUNK, DSUB), jnp.float32),
        pltpu.SemaphoreType.DMA,
    ],
)
def _sc_gather(idx_hbm, table_hbm, out_hbm, idx_v, rows_v, sem):
    wid = lax.axis_index("s") * NCORES + lax.axis_index("c")
    pltpu.sync_copy(idx_hbm.at[pl.ds(wid * NCHUNK, NCHUNK)], idx_v)
    copies = [
        pltpu.async_copy(table_hbm.at[idx_v.at[j]], rows_v.at[j], sem)
        for j in range(NCHUNK)
    ]
    for c in copies:
        c.wait()
    pltpu.sync_copy(rows_v, out_hbm.at[pl.ds(wid * NCHUNK, NCHUNK)])


@jax.jit
def kernel(vecs, codebook):
    idx = _tc_indices(vecs, codebook)
    idx3 = jnp.transpose(idx, (1, 0, 2)).reshape(NW * NCHUNK, CHUNK)
    table = codebook.reshape(PARTITION * CENTROIDS, DSUB)
    out = _sc_gather(idx3, table)
    return out.reshape(BATCH, EMBED)


# speedup vs baseline: 1.0152x; 1.0152x over previous
import functools

import jax
import jax.numpy as jnp
from jax import lax
from jax.experimental import pallas as pl
from jax.experimental.pallas import tpu as pltpu
from jax.experimental.pallas import tpu_sc as plsc

BATCH = 1024
EMBED = 768
PARTITION = 96
CENTROIDS = 256
DSUB = EMBED // PARTITION
PGROUP = 16
NGRP = PARTITION // PGROUP

_CONTRACT_T = (((1,), (1,)), ((), ()))

NCORES = 2
NSUB = 16
NW = NCORES * NSUB
ROWS = BATCH * PARTITION
CHUNK = 128
NCHUNK = ROWS // (NW * CHUNK)
BPW = BATCH // NW


def _dot(a, b, dn):
    return jax.lax.dot_general(a, b, dn, preferred_element_type=jnp.float32)


def _idx_body(v_ref, cb_ref, idx_ref, cbdh_ref, cbdl_ref):
    step = pl.program_id(0)

    @pl.when(step == 0)
    def _zero():
        cbdh_ref[...] = jnp.zeros_like(cbdh_ref)
        cbdl_ref[...] = jnp.zeros_like(cbdl_ref)

    for g in range(PGROUP):
        slab = cb_ref[g]
        hi = slab.astype(jnp.bfloat16)
        lo = (slab - hi.astype(jnp.float32)).astype(jnp.bfloat16)
        rows = slice(g * CENTROIDS, (g + 1) * CENTROIDS)
        cols = slice(g * DSUB, (g + 1) * DSUB)
        cbdh_ref[rows, cols] = hi
        cbdl_ref[rows, cols] = lo

    v = v_ref[...]
    vh = v.astype(jnp.bfloat16)
    vl = (v - vh.astype(jnp.float32)).astype(jnp.bfloat16)

    s = (_dot(vh, cbdh_ref[...], _CONTRACT_T) + _dot(vh, cbdl_ref[...], _CONTRACT_T)
         + _dot(vl, cbdh_ref[...], _CONTRACT_T))

    iota = lax.broadcasted_iota(jnp.int32, (BATCH, CENTROIDS), 1)
    base0 = (step * PGROUP) * CENTROIDS
    cols = []
    for g in range(PGROUP):
        c_p = cb_ref[g]
        csq = 0.5 * jnp.sum(c_p * c_p, axis=-1)
        sg = s[:, g * CENTROIDS:(g + 1) * CENTROIDS] - csq[None, :]
        m = jnp.max(sg, axis=1, keepdims=True)
        idx = jnp.min(jnp.where(sg == m, iota, CENTROIDS), axis=1,
                      keepdims=True)
        cols.append(idx + (base0 + g * CENTROIDS))
    idx_ref[0] = jnp.concatenate(cols, axis=1)


def _tc_indices(vecs, codebook):
    return pl.pallas_call(
        _idx_body,
        grid=(NGRP,),
        in_specs=[
            pl.BlockSpec((BATCH, PGROUP * DSUB), lambda i: (0, i)),
            pl.BlockSpec((PGROUP, CENTROIDS, DSUB), lambda i: (i, 0, 0)),
        ],
        out_specs=pl.BlockSpec((1, BATCH, PGROUP), lambda i: (i, 0, 0)),
        out_shape=jax.ShapeDtypeStruct((NGRP, BATCH, PGROUP), jnp.int32),
        scratch_shapes=[
            pltpu.VMEM((PGROUP * CENTROIDS, PGROUP * DSUB), jnp.bfloat16),
            pltpu.VMEM((PGROUP * CENTROIDS, PGROUP * DSUB), jnp.bfloat16),
        ],
    )(vecs, codebook)


@functools.partial(
    pl.kernel,
    mesh=plsc.VectorSubcoreMesh(core_axis_name="c", subcore_axis_name="s"),
    compiler_params=pltpu.CompilerParams(use_tc_tiling_on_sc=False),
    out_type=jax.ShapeDtypeStruct((NW * NCHUNK, CHUNK, DSUB), jnp.float32),
    scratch_types=[
        pltpu.VMEM((NGRP, BPW, PGROUP), jnp.int32),
        pltpu.VMEM((NCHUNK, CHUNK), jnp.int32),
        pltpu.VMEM((NCHUNK, CHUNK, DSUB), jnp.float32),
        pltpu.SemaphoreType.DMA,
    ],
)
def _sc_gather(idx_hbm, table_hbm, out_hbm, raw_v, idx_v, rows_v, sem):
    wid = lax.axis_index("s") * NCORES + lax.axis_index("c")
    b0 = wid * BPW
    pltpu.sync_copy(idx_hbm.at[:, pl.ds(b0, BPW), :], raw_v)
    for b in range(BPW):
        for st in range(NGRP):
            t = b * PARTITION + st * PGROUP
            idx_v[t // CHUNK, t % CHUNK:t % CHUNK + PGROUP] = raw_v[st, b, :]
    copies = [
        pltpu.async_copy(table_hbm.at[idx_v.at[j]], rows_v.at[j], sem)
        for j in range(NCHUNK)
    ]
    for c in copies:
        c.wait()
    pltpu.sync_copy(rows_v, out_hbm.at[pl.ds(wid * NCHUNK, NCHUNK)])


@jax.jit
def kernel(vecs, codebook):
    idx = _tc_indices(vecs, codebook)
    table = codebook.reshape(PARTITION * CENTROIDS, DSUB)
    out = _sc_gather(idx, table)
    return out.reshape(BATCH, EMBED)
